# baseline (speedup 1.0000x reference)
import functools

import jax
import jax.numpy as jnp
from jax import lax
from jax.experimental import pallas as pl
from jax.experimental.pallas import tpu as pltpu
from jax.experimental.pallas import tpu_sc as plsc

_VOCAB = 1000000
_EMB = 64
_ENC = 64
_BS = 512
_DEPTH = 5
_NNODES = 2 ** _DEPTH - 1

_NC = 2
_NS = 16
_NW = _NC * _NS

_TOK_PAD = 16384
_TOK_PER_W = _TOK_PAD // _NW
_REC_PER_TOK = _EMB
_CHUNK = 128
_CHUNKS_PER_W = _TOK_PER_W * _REC_PER_TOK // _CHUNK

def _sc_gather(table_flat, idx):
    mesh = plsc.VectorSubcoreMesh(core_axis_name="c", subcore_axis_name="s")

    @functools.partial(
        pl.kernel,
        out_type=jax.ShapeDtypeStruct((_TOK_PAD // 2, 2 * _EMB), jnp.float32),
        mesh=mesh,
        scratch_types=[
            pltpu.VMEM((_CHUNKS_PER_W, _CHUNK), jnp.int32),
            pltpu.VMEM((_CHUNKS_PER_W, _CHUNK), jnp.float32),
            pltpu.SemaphoreType.DMA,
        ],
    )
    def k(table_hbm, idx_hbm, out_hbm, idx_v, rows_v, sem):
        wid = lax.axis_index("s") * _NC + lax.axis_index("c")
        pltpu.sync_copy(idx_hbm.at[wid], idx_v)
        copies = []
        for j in range(_CHUNKS_PER_W):
            copies.append(
                pltpu.async_copy(table_hbm.at[idx_v.at[j]], rows_v.at[j], sem)
            )
        for c in copies:
            c.wait()
        pltpu.sync_copy(
            rows_v, out_hbm.at[pl.ds(wid * _CHUNKS_PER_W, _CHUNKS_PER_W)])

    return k(table_flat, idx)


def _tree_body(g_ref, wih_ref, whh_ref, bih_ref, bhh_ref, sw_ref,
               sb_ref, cw_ref, out_ref):
    wih = wih_ref[...]
    whh = whh_ref[...]
    bih = bih_ref[...]
    bhh = bhh_ref[...]
    sw = sw_ref[...]
    sb = sb_ref[...]
    cw = cw_ref[...]

    E = _ENC
    HB = _BS // 2
    h_prev = None
    acc = None
    for l in range(_DEPTH - 1, -1, -1):
        n = 1 << l
        start = (n - 1) * _BS // 2
        blk = g_ref[pl.ds(start, n * HB), :]
        ee = blk[:, :E].reshape(n, HB, E)
        eo = blk[:, E:].reshape(n, HB, E)
        emb = jnp.concatenate([ee, eo], axis=1).reshape(n * _BS, E)
        gi = jnp.dot(emb, wih, preferred_element_type=jnp.float32) + bih
        if l == _DEPTH - 1:
            gh = jnp.broadcast_to(bhh, (n * _BS, 3 * E))
            h0 = None
        else:
            hp = h_prev
            w1 = jnp.tanh(jnp.dot(hp, sw, preferred_element_type=jnp.float32) + sb)
            t4 = (w1 * cw).reshape(n, 2, _BS, E)
            s = jnp.tanh(jnp.sum(t4, axis=-1, keepdims=True))
            s0 = s[:, 0]
            s1 = s[:, 1]
            a0 = jax.nn.sigmoid(s0 - s1)
            a1 = jax.nn.sigmoid(s1 - s0)
            ch = hp.reshape(n, 2, _BS, E)
            h0 = (ch[:, 0] * a0 + ch[:, 1] * a1).reshape(n * _BS, E)
            gh = jnp.dot(h0, whh, preferred_element_type=jnp.float32) + bhh
        r = jax.nn.sigmoid(gi[:, :E] + gh[:, :E])
        z = jax.nn.sigmoid(gi[:, E:2 * E] + gh[:, E:2 * E])
        c = jnp.tanh(gi[:, 2 * E:] + r * gh[:, 2 * E:])
        if l == _DEPTH - 1:
            h = (1.0 - z) * c
        else:
            h = (1.0 - z) * c + z * h0
        lvl_max = jnp.max(h.reshape(n, _BS, E), axis=0)
        acc = lvl_max if acc is None else jnp.maximum(acc, lvl_max)
        h_prev = h
    out_ref[...] = acc.reshape(2, HB, E).transpose(1, 0, 2).reshape(_BS, E)


def _tc_compute(gathered, wih_t, whh_t, bih2, bhh2, sw, sb, cw_t,
                interpret=False):
    return pl.pallas_call(
        _tree_body,
        out_shape=jax.ShapeDtypeStruct((_BS, _ENC), jnp.float32),
        interpret=interpret,
    )(gathered, wih_t, whh_t, bih2, bhh2, sw, sb, cw_t)


def _flat_offsets(tokens):
    flat = tokens.astype(jnp.int32).T.reshape(-1)
    flat = jnp.concatenate(
        [flat, jnp.zeros((_TOK_PAD - _NNODES * _BS,), jnp.int32)])
    e = jnp.arange(_EMB, dtype=jnp.int32) * _VOCAB
    offs = flat[:, None] + e[None, :]
    return offs.reshape(_NW, _CHUNKS_PER_W, _CHUNK)


def kernel(tokens, embedding, W_ih, W_hh, b_ih, b_hh, sent_weight, sent_bias,
           context_weight):
    idx = _flat_offsets(tokens)
    gathered = _sc_gather(embedding.T.reshape(-1), idx)
    return _tc_compute(
        gathered,
        W_ih.T, W_hh.T,
        b_ih.reshape(1, -1), b_hh.reshape(1, -1),
        sent_weight, sent_bias,
        context_weight.reshape(1, -1),
    )


# speedup vs baseline: 6.6422x; 6.6422x over previous
import functools

import jax
import jax.numpy as jnp
from jax import lax
from jax.experimental import pallas as pl
from jax.experimental.pallas import tpu as pltpu
from jax.experimental.pallas import tpu_sc as plsc

_VOCAB = 1000000
_EMB = 64
_ENC = 64
_BS = 512
_DEPTH = 5
_NNODES = 2 ** _DEPTH - 1

_NC = 2
_NS = 16
_NW = _NC * _NS
_CHUNK = 128
_CHUNKS_PER_W = 4
_ROWS_PER_W = _CHUNK * _CHUNKS_PER_W
_ROWS_PAD = _NW * _ROWS_PER_W

_TR = 512
_H = 977 * _TR


def _detile_body(x1_ref, x2_ref, o_ref):
    o_ref[...] = jnp.concatenate([x1_ref[...].T, x2_ref[...].T], axis=1)


def _detile(table_t):
    grid = _H // _TR
    return pl.pallas_call(
        _detile_body,
        grid=(grid,),
        in_specs=[
            pl.BlockSpec((_EMB, _TR), lambda i: (0, i)),
            pl.BlockSpec((_EMB, _TR), lambda i: (0, i + _H // _TR)),
        ],
        out_specs=pl.BlockSpec((_TR, 2 * _EMB), lambda i: (i, 0)),
        out_shape=jax.ShapeDtypeStruct((_H, 2 * _EMB), jnp.float32),
    )(table_t, table_t)


def _sc_gather(table, idx):
    mesh = plsc.VectorSubcoreMesh(core_axis_name="c", subcore_axis_name="s")

    @functools.partial(
        pl.kernel,
        out_type=jax.ShapeDtypeStruct((_ROWS_PAD, 2 * _EMB), jnp.float32),
        mesh=mesh,
        scratch_types=[
            pltpu.VMEM((_CHUNKS_PER_W, _CHUNK), jnp.int32),
            pltpu.VMEM((_ROWS_PER_W, 2 * _EMB), jnp.float32),
            pltpu.SemaphoreType.DMA,
        ],
    )
    def k(table_hbm, idx_hbm, out_hbm, idx_v, rows_v, sem):
        wid = lax.axis_index("s") * _NC + lax.axis_index("c")
        pltpu.sync_copy(idx_hbm.at[wid], idx_v)
        copies = []
        for j in range(_CHUNKS_PER_W):
            copies.append(
                pltpu.async_copy(
                    table_hbm.at[idx_v.at[j]],
                    rows_v.at[pl.ds(j * _CHUNK, _CHUNK)],
                    sem,
                )
            )
        for c in copies:
            c.wait()
        pltpu.sync_copy(rows_v, out_hbm.at[pl.ds(wid * _ROWS_PER_W, _ROWS_PER_W)])

    return k(table, idx)


def _tree_body(g_ref, par_ref, wih_ref, whh_ref, bih_ref, bhh_ref, sw_ref,
               sb_ref, cw_ref, out_ref):
    wih = wih_ref[...]
    whh = whh_ref[...]
    bih = bih_ref[...]
    bhh = bhh_ref[...]
    sw = sw_ref[...]
    sb = sb_ref[...]
    cw = cw_ref[...]

    E = _ENC
    h_prev = None
    acc = None
    for l in range(_DEPTH - 1, -1, -1):
        n = 1 << l
        start = (n - 1) * _BS
        g = g_ref[pl.ds(start, n * _BS), :]
        par = par_ref[pl.ds(start, n * _BS), :]
        emb = jnp.where(par == 1, g[:, E:], g[:, :E])
        gi = jnp.dot(emb, wih, preferred_element_type=jnp.float32) + bih
        if l == _DEPTH - 1:
            gh = jnp.broadcast_to(bhh, (n * _BS, 3 * E))
            h0 = None
        else:
            hp = h_prev
            w1 = jnp.tanh(jnp.dot(hp, sw, preferred_element_type=jnp.float32) + sb)
            t4 = (w1 * cw).reshape(n, 2, _BS, E)
            s = jnp.tanh(jnp.sum(t4, axis=-1, keepdims=True))
            s0 = s[:, 0]
            s1 = s[:, 1]
            a0 = jax.nn.sigmoid(s0 - s1)
            a1 = jax.nn.sigmoid(s1 - s0)
            ch = hp.reshape(n, 2, _BS, E)
            h0 = (ch[:, 0] * a0 + ch[:, 1] * a1).reshape(n * _BS, E)
            gh = jnp.dot(h0, whh, preferred_element_type=jnp.float32) + bhh
        r = jax.nn.sigmoid(gi[:, :E] + gh[:, :E])
        z = jax.nn.sigmoid(gi[:, E:2 * E] + gh[:, E:2 * E])
        c = jnp.tanh(gi[:, 2 * E:] + r * gh[:, 2 * E:])
        if l == _DEPTH - 1:
            h = (1.0 - z) * c
        else:
            h = (1.0 - z) * c + z * h0
        lvl_max = jnp.max(h.reshape(n, _BS, E), axis=0)
        acc = lvl_max if acc is None else jnp.maximum(acc, lvl_max)
        h_prev = h
    out_ref[...] = acc


def _tc_compute(gathered, parity, wih_t, whh_t, bih2, bhh2, sw, sb, cw_t,
                interpret=False):
    return pl.pallas_call(
        _tree_body,
        out_shape=jax.ShapeDtypeStruct((_BS, _ENC), jnp.float32),
        interpret=interpret,
    )(gathered, parity, wih_t, whh_t, bih2, bhh2, sw, sb, cw_t)


def kernel(tokens, embedding, W_ih, W_hh, b_ih, b_hh, sent_weight, sent_bias,
           context_weight):
    flat = tokens.astype(jnp.int32).T.reshape(-1)
    flat = jnp.concatenate(
        [flat, jnp.zeros((_ROWS_PAD - _NNODES * _BS,), jnp.int32)])
    par = (flat >= _H).astype(jnp.int32)
    idx = (flat - par * _H).reshape(_NW, _CHUNKS_PER_W, _CHUNK)
    parity = par.reshape(_ROWS_PAD, 1)
    table = _detile(embedding.T)
    gathered = _sc_gather(table, idx)
    return _tc_compute(
        gathered, parity,
        W_ih.T, W_hh.T,
        b_ih.reshape(1, -1), b_hh.reshape(1, -1),
        sent_weight, sent_bias,
        context_weight.reshape(1, -1),
    )


# speedup vs baseline: 8.2730x; 1.2455x over previous
import functools

import jax
import jax.numpy as jnp
from jax import lax
from jax.experimental import pallas as pl
from jax.experimental.pallas import tpu as pltpu
from jax.experimental.pallas import tpu_sc as plsc

_VOCAB = 1000000
_EMB = 64
_ENC = 64
_BS = 512
_DEPTH = 5
_NNODES = 2 ** _DEPTH - 1

_NC = 2
_NS = 16
_NW = _NC * _NS
_CHUNK = 128
_CHUNKS_PER_W = 4
_ROWS_PER_W = _CHUNK * _CHUNKS_PER_W
_ROWS_PAD = _NW * _ROWS_PER_W


def _sc_gather(table, idx):
    mesh = plsc.VectorSubcoreMesh(core_axis_name="c", subcore_axis_name="s")

    @functools.partial(
        pl.kernel,
        out_type=jax.ShapeDtypeStruct((_ROWS_PAD, 2 * _EMB), jnp.float32),
        mesh=mesh,
        scratch_types=[
            pltpu.VMEM((_CHUNKS_PER_W, _CHUNK), jnp.int32),
            pltpu.VMEM((_ROWS_PER_W, 2 * _EMB), jnp.float32),
            pltpu.SemaphoreType.DMA,
        ],
    )
    def k(table_hbm, idx_hbm, out_hbm, idx_v, rows_v, sem):
        wid = lax.axis_index("s") * _NC + lax.axis_index("c")
        pltpu.sync_copy(idx_hbm.at[wid], idx_v)
        copies = []
        for j in range(_CHUNKS_PER_W):
            copies.append(
                pltpu.async_copy(
                    table_hbm.at[idx_v.at[j]],
                    rows_v.at[pl.ds(j * _CHUNK, _CHUNK)],
                    sem,
                )
            )
        for c in copies:
            c.wait()
        pltpu.sync_copy(rows_v, out_hbm.at[pl.ds(wid * _ROWS_PER_W, _ROWS_PER_W)])

    return k(table, idx)


def _tree_body(g_ref, par_ref, wih_ref, whh_ref, bih_ref, bhh_ref, sw_ref,
               sb_ref, cw_ref, out_ref):
    wih = wih_ref[...]
    whh = whh_ref[...]
    bih = bih_ref[...]
    bhh = bhh_ref[...]
    sw = sw_ref[...]
    sb = sb_ref[...]
    cw = cw_ref[...]

    E = _ENC
    h_prev = None
    acc = None
    for l in range(_DEPTH - 1, -1, -1):
        n = 1 << l
        start = (n - 1) * _BS
        g = g_ref[pl.ds(start, n * _BS), :]
        par = par_ref[pl.ds(start, n * _BS), :]
        emb = jnp.where(par == 1, g[:, E:], g[:, :E])
        gi = jnp.dot(emb, wih, preferred_element_type=jnp.float32) + bih
        if l == _DEPTH - 1:
            gh = jnp.broadcast_to(bhh, (n * _BS, 3 * E))
            h0 = None
        else:
            hp = h_prev
            w1 = jnp.tanh(jnp.dot(hp, sw, preferred_element_type=jnp.float32) + sb)
            t4 = (w1 * cw).reshape(n, 2, _BS, E)
            s = jnp.tanh(jnp.sum(t4, axis=-1, keepdims=True))
            s0 = s[:, 0]
            s1 = s[:, 1]
            a0 = jax.nn.sigmoid(s0 - s1)
            a1 = jax.nn.sigmoid(s1 - s0)
            ch = hp.reshape(n, 2, _BS, E)
            h0 = (ch[:, 0] * a0 + ch[:, 1] * a1).reshape(n * _BS, E)
            gh = jnp.dot(h0, whh, preferred_element_type=jnp.float32) + bhh
        r = jax.nn.sigmoid(gi[:, :E] + gh[:, :E])
        z = jax.nn.sigmoid(gi[:, E:2 * E] + gh[:, E:2 * E])
        c = jnp.tanh(gi[:, 2 * E:] + r * gh[:, 2 * E:])
        if l == _DEPTH - 1:
            h = (1.0 - z) * c
        else:
            h = (1.0 - z) * c + z * h0
        lvl_max = jnp.max(h.reshape(n, _BS, E), axis=0)
        acc = lvl_max if acc is None else jnp.maximum(acc, lvl_max)
        h_prev = h
    out_ref[...] = acc


def _tc_compute(gathered, parity, wih_t, whh_t, bih2, bhh2, sw, sb, cw_t,
                interpret=False):
    return pl.pallas_call(
        _tree_body,
        out_shape=jax.ShapeDtypeStruct((_BS, _ENC), jnp.float32),
        interpret=interpret,
    )(gathered, parity, wih_t, whh_t, bih2, bhh2, sw, sb, cw_t)


def kernel(tokens, embedding, W_ih, W_hh, b_ih, b_hh, sent_weight, sent_bias,
           context_weight):
    flat = tokens.astype(jnp.int32).T.reshape(-1)
    flat = jnp.concatenate(
        [flat, jnp.zeros((_ROWS_PAD - _NNODES * _BS,), jnp.int32)])
    idx = flat.reshape(_NW, _CHUNKS_PER_W, _CHUNK)
    parity = jnp.zeros((_ROWS_PAD, 1), jnp.int32)
    table = jnp.pad(embedding, ((0, 0), (0, _EMB)))
    gathered = _sc_gather(table, idx)
    return _tc_compute(
        gathered, parity,
        W_ih.T, W_hh.T,
        b_ih.reshape(1, -1), b_hh.reshape(1, -1),
        sent_weight, sent_bias,
        context_weight.reshape(1, -1),
    )
